# baseline (speedup 1.0000x reference)
import functools

import jax
import jax.numpy as jnp
from jax.experimental import pallas as pl
from jax.experimental.pallas import tpu as pltpu

_DELTA_V = 0.5
_DELTA_D = 1.5
_ALPHA = 1.0
_BETA = 1.0
_GAMMA = 0.001
_S = 16


def _dl_body(emb_ref, msk_ref, out_ref,
             emb_save, oh_save, sums, counts, vacc, aug, means_s,
             present_s, kscal, totals,
             *, S, nb, B, P):
    b = pl.program_id(0)
    i = pl.program_id(1)
    E = emb_ref.shape[1]

    @pl.when(jnp.logical_and(b == 0, i == 0))
    def _init_totals():
        totals[0] = 0.0
        totals[1] = 0.0
        totals[2] = 0.0
        totals[3] = 0.0

    @pl.when(jnp.logical_and(b > 0, i == 0))
    def _make_aug():
        cnt_row = counts[...]
        cc_row = jnp.maximum(cnt_row, 1.0)
        pres_row = (cnt_row > 0.0).astype(jnp.float32)
        K = jnp.sum(pres_row)
        w_row = pres_row / (cc_row * jnp.maximum(K, 1.0))
        w_hi = w_row.astype(jnp.bfloat16).astype(jnp.float32)
        w_lo = w_row - w_hi
        m_t = sums[...] / cc_row
        aug[...] = jnp.concatenate(
            [m_t, w_hi, w_lo], axis=0).astype(jnp.bfloat16)
        means_s[...] = m_t.T
        present_s[...] = pres_row.reshape(S, 1)
        kscal[0] = K

    @pl.when(i == 0)
    def _zero_row():
        vacc[...] = jnp.zeros_like(vacc)
        sums[...] = jnp.zeros_like(sums)
        counts[...] = jnp.zeros_like(counts)

    @pl.when(b > 0)
    def _pass2():
        x = emb_save[:, pl.ds(i * P, P)]
        ohb = oh_save[:, pl.ds(i * P, P)]
        pix = jnp.dot(aug[...], ohb,
                      preferred_element_type=jnp.float32)
        d = x - pix[:E, :]
        ones_e = jnp.ones((1, E), jnp.float32)
        pd2 = jnp.dot(ones_e, d * d,
                      preferred_element_type=jnp.float32)
        pd = jnp.sqrt(jnp.maximum(pd2, 1e-24))
        vt = jnp.maximum(pd - _DELTA_V, 0.0) ** 2
        wp = pix[E:E + 1, :] + pix[E + 1:E + 2, :]
        vacc[...] += wp * vt

    @pl.when(b < B)
    def _pass1():
        x = emb_ref[0].reshape(E, P)
        seg = msk_ref[0].reshape(1, P)
        sids = jax.lax.broadcasted_iota(jnp.int32, (S, P), 0) + 1
        ohb = (seg == sids).astype(jnp.bfloat16)
        emb_save[:, pl.ds(i * P, P)] = x
        oh_save[:, pl.ds(i * P, P)] = ohb
        sums[...] += jax.lax.dot_general(
            x.astype(jnp.bfloat16), ohb, (((1,), (1,)), ((), ())),
            preferred_element_type=jnp.float32)
        ones_p = jnp.ones((1, P), jnp.bfloat16)
        counts[...] += jax.lax.dot_general(
            ones_p, ohb, (((1,), (1,)), ((), ())),
            preferred_element_type=jnp.float32)

    @pl.when(jnp.logical_and(b > 0, i == nb - 1))
    def _finish_batch():
        K = kscal[0]
        safe_K = jnp.maximum(K, 1.0)
        batch_var = jnp.sum(vacc[...])

        present = present_s[...]
        m = means_s[...]
        diff = m[:, None, :] - m[None, :, :]
        d2 = jnp.sum(diff * diff, axis=-1)
        dmat = jnp.sqrt(jnp.maximum(d2, 1e-24))
        ii = jax.lax.broadcasted_iota(jnp.int32, (S, S), 0)
        jj = jax.lax.broadcasted_iota(jnp.int32, (S, S), 1)
        tri = (ii < jj).astype(jnp.float32)
        pmask = present * present.reshape(1, S)
        hinge = jnp.maximum(2.0 * _DELTA_D - dmat, 0.0) ** 2
        n_pairs = K * (K - 1.0) * 0.5
        batch_dist = jnp.sum(tri * pmask * hinge) / jnp.maximum(n_pairs, 1.0)
        batch_dist = jnp.where(K > 1.0, batch_dist, 0.0)

        mu2 = jnp.sum(m * m, axis=1, keepdims=True)
        batch_reg = jnp.sum(
            present * jnp.sqrt(jnp.maximum(mu2, 1e-24))) / safe_K

        has_fg = (K > 0.0).astype(jnp.float32)
        totals[0] = totals[0] + has_fg * batch_var
        totals[1] = totals[1] + has_fg * batch_dist
        totals[2] = totals[2] + has_fg * batch_reg
        totals[3] = totals[3] + has_fg

    @pl.when(jnp.logical_and(b == B, i == nb - 1))
    def _finish_all():
        valid = totals[3]
        denom = jnp.maximum(valid, 1.0)
        tv = jnp.where(valid > 0.0, totals[0] / denom, totals[0])
        td = jnp.where(valid > 0.0, totals[1] / denom, totals[1])
        tr = jnp.where(valid > 0.0, totals[2] / denom, totals[2])
        out_ref[0] = _ALPHA * tv + _BETA * td + _GAMMA * tr
        out_ref[1] = tv
        out_ref[2] = td
        out_ref[3] = tr


def kernel(embeddings, instance_masks):
    B, E, H, W = embeddings.shape
    N = H * W
    RH = H if N < 32768 else max(32768 // W, 8)
    P = RH * W
    nb = H // RH

    def _emb_map(b, i):
        return (jnp.minimum(b, B - 1), 0, jnp.where(b < B, i, 0), 0)

    def _msk_map(b, i):
        return (jnp.minimum(b, B - 1), jnp.where(b < B, i, 0), 0)

    body = functools.partial(_dl_body, S=_S, nb=nb, B=B, P=P)
    out = pl.pallas_call(
        body,
        grid=(B + 1, nb),
        in_specs=[
            pl.BlockSpec((1, E, RH, W), _emb_map),
            pl.BlockSpec((1, RH, W), _msk_map),
        ],
        out_specs=pl.BlockSpec(
            (4,), lambda b, i: (0,), memory_space=pltpu.SMEM),
        out_shape=jax.ShapeDtypeStruct((4,), jnp.float32),
        scratch_shapes=[
            pltpu.VMEM((E, N), jnp.float32),
            pltpu.VMEM((_S, N), jnp.bfloat16),
            pltpu.VMEM((E, _S), jnp.float32),
            pltpu.VMEM((1, _S), jnp.float32),
            pltpu.VMEM((1, P), jnp.float32),
            pltpu.VMEM((E + 2, _S), jnp.bfloat16),
            pltpu.VMEM((_S, E), jnp.float32),
            pltpu.VMEM((_S, 1), jnp.float32),
            pltpu.SMEM((1,), jnp.float32),
            pltpu.SMEM((4,), jnp.float32),
        ],
    )(embeddings, instance_masks)
    return {"loss": out[0], "var_loss": out[1],
            "dist_loss": out[2], "reg_loss": out[3]}


# speedup vs baseline: 1.1921x; 1.1921x over previous
import functools

import jax
import jax.numpy as jnp
from jax.experimental import pallas as pl
from jax.experimental.pallas import tpu as pltpu

_DELTA_V = 0.5
_DELTA_D = 1.5
_ALPHA = 1.0
_BETA = 1.0
_GAMMA = 0.001
_S = 16


def _dl_body(emb_ref, msk_ref, out_ref,
             emb_save, oh_save, sums, counts, vacc, aug, means_s,
             present_s, kscal, totals,
             *, S, nb, B, P):
    b = pl.program_id(0)
    i = pl.program_id(1)
    E = emb_ref.shape[1]

    @pl.when(jnp.logical_and(b == 0, i == 0))
    def _init_totals():
        totals[0] = 0.0
        totals[1] = 0.0
        totals[2] = 0.0
        totals[3] = 0.0

    @pl.when(jnp.logical_and(b > 0, i == 0))
    def _make_aug():
        cnt_row = counts[...].reshape(1, S)
        cc_row = jnp.maximum(cnt_row, 1.0)
        pres_row = (cnt_row > 0.0).astype(jnp.float32)
        K = jnp.sum(pres_row)
        w_row = pres_row / (cc_row * jnp.maximum(K, 1.0))
        w_hi = w_row.astype(jnp.bfloat16).astype(jnp.float32)
        w_lo = w_row - w_hi
        m_t = sums[...] / cc_row
        aug[...] = jnp.concatenate(
            [m_t, w_hi, w_lo], axis=0).astype(jnp.bfloat16)
        means_s[...] = m_t.T
        present_s[...] = pres_row.reshape(S, 1)
        kscal[0] = K

    @pl.when(i == 0)
    def _zero_row():
        vacc[...] = jnp.zeros_like(vacc)
        sums[...] = jnp.zeros_like(sums)
        counts[...] = jnp.zeros_like(counts)

    @pl.when(b > 0)
    def _pass2():
        x = emb_save[:, pl.ds(i * P, P)]
        ohb = oh_save[:, pl.ds(i * P, P)]
        pix = jnp.dot(aug[...], ohb,
                      preferred_element_type=jnp.float32)
        d = x - pix[:E, :]
        ones_e = jnp.ones((1, E), jnp.float32)
        pd2 = jnp.dot(ones_e, d * d,
                      preferred_element_type=jnp.float32)
        pd = jnp.sqrt(jnp.maximum(pd2, 1e-24))
        vt = jnp.maximum(pd - _DELTA_V, 0.0) ** 2
        wp = pix[E:E + 1, :] + pix[E + 1:E + 2, :]
        vacc[...] += wp * vt

    @pl.when(b < B)
    def _pass1():
        x = emb_ref[0].reshape(E, P)
        seg = msk_ref[0].reshape(1, P)
        sids = jax.lax.broadcasted_iota(jnp.int32, (S, P), 0) + 1
        oh = (seg == sids).astype(jnp.float32)
        emb_save[:, pl.ds(i * P, P)] = x
        oh_save[:, pl.ds(i * P, P)] = oh.astype(jnp.bfloat16)
        sums[...] += jax.lax.dot_general(
            x, oh, (((1,), (1,)), ((), ())),
            preferred_element_type=jnp.float32)
        counts[...] += jnp.sum(oh, axis=1, keepdims=True)

    @pl.when(jnp.logical_and(b > 0, i == nb - 1))
    def _finish_batch():
        K = kscal[0]
        safe_K = jnp.maximum(K, 1.0)
        batch_var = jnp.sum(vacc[...])

        present = present_s[...]
        m = means_s[...]
        diff = m[:, None, :] - m[None, :, :]
        d2 = jnp.sum(diff * diff, axis=-1)
        dmat = jnp.sqrt(jnp.maximum(d2, 1e-24))
        ii = jax.lax.broadcasted_iota(jnp.int32, (S, S), 0)
        jj = jax.lax.broadcasted_iota(jnp.int32, (S, S), 1)
        tri = (ii < jj).astype(jnp.float32)
        pmask = present * present.reshape(1, S)
        hinge = jnp.maximum(2.0 * _DELTA_D - dmat, 0.0) ** 2
        n_pairs = K * (K - 1.0) * 0.5
        batch_dist = jnp.sum(tri * pmask * hinge) / jnp.maximum(n_pairs, 1.0)
        batch_dist = jnp.where(K > 1.0, batch_dist, 0.0)

        mu2 = jnp.sum(m * m, axis=1, keepdims=True)
        batch_reg = jnp.sum(
            present * jnp.sqrt(jnp.maximum(mu2, 1e-24))) / safe_K

        has_fg = (K > 0.0).astype(jnp.float32)
        totals[0] = totals[0] + has_fg * batch_var
        totals[1] = totals[1] + has_fg * batch_dist
        totals[2] = totals[2] + has_fg * batch_reg
        totals[3] = totals[3] + has_fg

    @pl.when(jnp.logical_and(b == B, i == nb - 1))
    def _finish_all():
        valid = totals[3]
        denom = jnp.maximum(valid, 1.0)
        tv = jnp.where(valid > 0.0, totals[0] / denom, totals[0])
        td = jnp.where(valid > 0.0, totals[1] / denom, totals[1])
        tr = jnp.where(valid > 0.0, totals[2] / denom, totals[2])
        out_ref[0] = _ALPHA * tv + _BETA * td + _GAMMA * tr
        out_ref[1] = tv
        out_ref[2] = td
        out_ref[3] = tr


def kernel(embeddings, instance_masks):
    B, E, H, W = embeddings.shape
    N = H * W
    RH = H if N < 32768 else max(32768 // W, 8)
    P = RH * W
    nb = H // RH

    def _emb_map(b, i):
        return (jnp.minimum(b, B - 1), 0, jnp.where(b < B, i, 0), 0)

    def _msk_map(b, i):
        return (jnp.minimum(b, B - 1), jnp.where(b < B, i, 0), 0)

    body = functools.partial(_dl_body, S=_S, nb=nb, B=B, P=P)
    out = pl.pallas_call(
        body,
        grid=(B + 1, nb),
        in_specs=[
            pl.BlockSpec((1, E, RH, W), _emb_map),
            pl.BlockSpec((1, RH, W), _msk_map),
        ],
        out_specs=pl.BlockSpec(
            (4,), lambda b, i: (0,), memory_space=pltpu.SMEM),
        out_shape=jax.ShapeDtypeStruct((4,), jnp.float32),
        scratch_shapes=[
            pltpu.VMEM((E, N), jnp.float32),
            pltpu.VMEM((_S, N), jnp.bfloat16),
            pltpu.VMEM((E, _S), jnp.float32),
            pltpu.VMEM((_S, 1), jnp.float32),
            pltpu.VMEM((1, P), jnp.float32),
            pltpu.VMEM((E + 2, _S), jnp.bfloat16),
            pltpu.VMEM((_S, E), jnp.float32),
            pltpu.VMEM((_S, 1), jnp.float32),
            pltpu.SMEM((1,), jnp.float32),
            pltpu.SMEM((4,), jnp.float32),
        ],
    )(embeddings, instance_masks)
    return {"loss": out[0], "var_loss": out[1],
            "dist_loss": out[2], "reg_loss": out[3]}
